# Initial kernel scaffold; baseline (speedup 1.0000x reference)
import jax
import jax.numpy as jnp
from jax.experimental import pallas as pl


def kernel(x_graph, action, temp_img, edge_index, batch, W1, att_src1, att_dst1, b1, W2, att_src2, att_dst2, b2, W_fc, b_fc):
    raise NotImplementedError("write your pallas kernel here")


# speedup vs baseline: 1.7113x; 1.7113x over previous
import functools

import jax
import jax.numpy as jnp
from jax.experimental import pallas as pl
from jax.experimental.pallas import tpu as pltpu

N_NODES = 10000
N_PAD = 10240
ROW_BLK = 2048
C = 512
NUM_GRAPHS = 8


def _leaky(x, slope):
    return jnp.where(x > 0, x, slope * x)


def _proj_body(x_ref, w_ref, asrc_ref, adst_ref, xp_ref, as_ref, ad_ref):
    xp = jnp.dot(x_ref[...], w_ref[...], preferred_element_type=jnp.float32)
    xp_ref[...] = xp
    as_ref[...] = xp @ asrc_ref[...]
    ad_ref[...] = xp @ adst_ref[...]


def _project(x, W, a_src, a_dst):
    n, k = x.shape
    c = W.shape[1]
    grid = (n // ROW_BLK,)
    xp, as_, ad_ = pl.pallas_call(
        _proj_body,
        grid=grid,
        in_specs=[
            pl.BlockSpec((ROW_BLK, k), lambda i: (i, 0)),
            pl.BlockSpec((k, c), lambda i: (0, 0)),
            pl.BlockSpec((c, 128), lambda i: (0, 0)),
            pl.BlockSpec((c, 128), lambda i: (0, 0)),
        ],
        out_specs=[
            pl.BlockSpec((ROW_BLK, c), lambda i: (i, 0)),
            pl.BlockSpec((ROW_BLK, 128), lambda i: (i, 0)),
            pl.BlockSpec((ROW_BLK, 128), lambda i: (i, 0)),
        ],
        out_shape=[
            jax.ShapeDtypeStruct((n, c), jnp.float32),
            jax.ShapeDtypeStruct((n, 128), jnp.float32),
            jax.ShapeDtypeStruct((n, 128), jnp.float32),
        ],
    )(x, W, a_src, a_dst)
    return xp, as_[:, 0], ad_[:, 0]


def _epi_proj_body(u_ref, den_ref, xp_ref, as_ref, ad_ref, b_ref, w_ref,
                   asrc_ref, adst_ref, h_ref, xp2_ref, as2_ref, ad2_ref):
    wself = jnp.exp(_leaky(as_ref[...] + ad_ref[...], 0.2))
    num = u_ref[...] + wself[:, 0][:, None] * xp_ref[...]
    den = den_ref[...][:, 0] + wself[:, 0]
    h = _leaky(num / den[:, None] + b_ref[...], 0.01)
    h_ref[...] = h
    xp2 = jnp.dot(h, w_ref[...], preferred_element_type=jnp.float32)
    xp2_ref[...] = xp2
    as2_ref[...] = xp2 @ asrc_ref[...]
    ad2_ref[...] = xp2 @ adst_ref[...]


def _epilogue_project(u, denom, xp, as_, ad_, b, W, a_src, a_dst):
    n, c = u.shape
    c2 = W.shape[1]
    grid = (n // ROW_BLK,)
    h, xp2, as2, ad2 = pl.pallas_call(
        _epi_proj_body,
        grid=grid,
        in_specs=[
            pl.BlockSpec((ROW_BLK, c), lambda i: (i, 0)),
            pl.BlockSpec((ROW_BLK, 128), lambda i: (i, 0)),
            pl.BlockSpec((ROW_BLK, c), lambda i: (i, 0)),
            pl.BlockSpec((ROW_BLK, 128), lambda i: (i, 0)),
            pl.BlockSpec((ROW_BLK, 128), lambda i: (i, 0)),
            pl.BlockSpec((1, c), lambda i: (0, 0)),
            pl.BlockSpec((c, c2), lambda i: (0, 0)),
            pl.BlockSpec((c2, 128), lambda i: (0, 0)),
            pl.BlockSpec((c2, 128), lambda i: (0, 0)),
        ],
        out_specs=[
            pl.BlockSpec((ROW_BLK, c), lambda i: (i, 0)),
            pl.BlockSpec((ROW_BLK, c2), lambda i: (i, 0)),
            pl.BlockSpec((ROW_BLK, 128), lambda i: (i, 0)),
            pl.BlockSpec((ROW_BLK, 128), lambda i: (i, 0)),
        ],
        out_shape=[
            jax.ShapeDtypeStruct((n, c), jnp.float32),
            jax.ShapeDtypeStruct((n, c2), jnp.float32),
            jax.ShapeDtypeStruct((n, 128), jnp.float32),
            jax.ShapeDtypeStruct((n, 128), jnp.float32),
        ],
    )(u, denom, xp, as_, ad_, b.reshape(1, c), W, a_src, a_dst)
    return h, xp2, as2[:, 0], ad2[:, 0]


def _final_body(u_ref, den_ref, xp_ref, as_ref, ad_ref, b_ref, wfc_ref,
                bfc_ref, onehot_ref, out_ref, cnt_ref):
    i = pl.program_id(0)
    wself = jnp.exp(_leaky(as_ref[...] + ad_ref[...], 0.2))
    num = u_ref[...] + wself[:, 0][:, None] * xp_ref[...]
    den = den_ref[...][:, 0] + wself[:, 0]
    h = _leaky(num / den[:, None] + b_ref[...], 0.01)
    y = _leaky(jnp.dot(h, wfc_ref[...], preferred_element_type=jnp.float32)
               + bfc_ref[...], 0.01)
    oh = onehot_ref[...]
    part = jnp.dot(oh, y, preferred_element_type=jnp.float32)
    cnt = jnp.sum(oh, axis=1, keepdims=True)

    @pl.when(i == 0)
    def _init():
        out_ref[...] = jnp.zeros_like(out_ref)
        cnt_ref[...] = jnp.zeros_like(cnt_ref)

    out_ref[...] += part
    cnt_ref[...] += cnt


def _final(u, denom, xp, as_, ad_, b, W_fc, b_fc, onehot):
    n, c = u.shape
    c2 = W_fc.shape[1]
    grid = (n // ROW_BLK,)
    out, cnt = pl.pallas_call(
        _final_body,
        grid=grid,
        in_specs=[
            pl.BlockSpec((ROW_BLK, c), lambda i: (i, 0)),
            pl.BlockSpec((ROW_BLK, 128), lambda i: (i, 0)),
            pl.BlockSpec((ROW_BLK, c), lambda i: (i, 0)),
            pl.BlockSpec((ROW_BLK, 128), lambda i: (i, 0)),
            pl.BlockSpec((ROW_BLK, 128), lambda i: (i, 0)),
            pl.BlockSpec((1, c), lambda i: (0, 0)),
            pl.BlockSpec((c, c2), lambda i: (0, 0)),
            pl.BlockSpec((1, c2), lambda i: (0, 0)),
            pl.BlockSpec((NUM_GRAPHS, ROW_BLK), lambda i: (0, i)),
        ],
        out_specs=[
            pl.BlockSpec((NUM_GRAPHS, c2), lambda i: (0, 0)),
            pl.BlockSpec((NUM_GRAPHS, 128), lambda i: (0, 0)),
        ],
        out_shape=[
            jax.ShapeDtypeStruct((NUM_GRAPHS, c2), jnp.float32),
            jax.ShapeDtypeStruct((NUM_GRAPHS, 128), jnp.float32),
        ],
    )(u, denom, xp, as_, ad_, b.reshape(1, c), W_fc, b_fc.reshape(1, c2),
      onehot)
    return out / jnp.maximum(cnt[:, :1], 1.0)


def _edge_phase(src, dst, as_, ad_, xp):
    n = xp.shape[0]
    alpha = _leaky(as_[src] + ad_[dst], 0.2)
    we = jnp.exp(alpha)
    denom = jax.ops.segment_sum(we, dst, num_segments=n)
    u = jax.ops.segment_sum(xp[src] * we[:, None], dst, num_segments=n)
    return u, denom


def _pad_rows(a, n_pad):
    pad = n_pad - a.shape[0]
    return jnp.pad(a, ((0, pad),) + ((0, 0),) * (a.ndim - 1))


def kernel(x_graph, action, temp_img, edge_index, batch, W1, att_src1,
           att_dst1, b1, W2, att_src2, att_dst2, b2, W_fc, b_fc):
    n = x_graph.shape[0]
    x = jnp.concatenate([x_graph, action, temp_img], axis=1)
    x = _pad_rows(x, N_PAD)
    src, dst = edge_index[0], edge_index[1]

    def col(v):
        return jnp.pad(v.reshape(-1, 1), ((0, 0), (0, 127)))

    def wide(v):
        return _pad_rows(jnp.pad(v[:, None], ((0, 0), (0, 127))), N_PAD)

    xp1, as1, ad1 = _project(x, W1, col(att_src1), col(att_dst1))
    u1, den1 = _edge_phase(src, dst, as1[:n], ad1[:n], xp1[:n])
    h1, xp2, as2, ad2 = _epilogue_project(
        _pad_rows(u1, N_PAD), wide(den1), xp1, wide(as1), wide(ad1), b1, W2,
        col(att_src2), col(att_dst2))
    u2, den2 = _edge_phase(src, dst, as2[:n], ad2[:n], xp2[:n])

    gids = jnp.arange(NUM_GRAPHS, dtype=jnp.int32)
    onehot = (batch[None, :] == gids[:, None]).astype(jnp.float32)
    onehot = jnp.pad(onehot, ((0, 0), (0, N_PAD - n)))

    return _final(_pad_rows(u2, N_PAD), wide(den2), xp2, wide(as2),
                  wide(ad2), b2, W_fc, b_fc, onehot)
